# baseline (speedup 1.0000x reference)
import functools

import jax
import jax.numpy as jnp
from jax import lax
from jax.experimental import pallas as pl
from jax.experimental.pallas import tpu as pltpu
from jax.experimental.pallas import tpu_sc as plsc

VOCAB = 100000
DIM = 32
WIN = 5
BATCH = 1024
H1 = 80
H2 = 40
NIDX = BATCH * WIN

VT = 1024
GRID = (VOCAB + VT - 1) // VT
NBUF = 8
TAIL = VOCAB - (GRID - 1) * VT
KP = 128

_NC, _NS = 2, 16
_NW = _NC * _NS
_BPW = NIDX // _NW
_CH = 2
_CW = _BPW // _CH


@functools.cache
def _make_sc_gather():
    @functools.partial(
        pl.kernel,
        out_type=jax.ShapeDtypeStruct((NIDX, DIM), jnp.float32),
        mesh=plsc.VectorSubcoreMesh(core_axis_name="c", subcore_axis_name="s"),
        scratch_types=[
            pltpu.VMEM((_CH, _CW), jnp.int32),
            pltpu.VMEM((_CH, _CW, DIM), jnp.float32),
            pltpu.SemaphoreType.DMA,
        ],
        compiler_params=pltpu.CompilerParams(use_tc_tiling_on_sc=False),
    )
    def _sc_gather(table_hbm, idx_hbm, out_hbm, idx_v, rows_v, sem):
        wid = lax.axis_index("s") * _NC + lax.axis_index("c")
        base = wid * _BPW
        copies = []
        for j in range(_CH):
            pltpu.sync_copy(idx_hbm.at[wid * _CH + j], idx_v.at[j])
            copies.append(
                pltpu.async_copy(table_hbm.at[idx_v.at[j]], rows_v.at[j], sem))
        for j in range(_CH):
            copies[j].wait()
            pltpu.sync_copy(rows_v.at[j], out_hbm.at[pl.ds(base + j * _CW, _CW)])

    return _sc_gather


def _mlp_body(xe_ref, w1_ref, b1_ref, w2_ref, b2_ref, w3_ref, b3_ref,
              out_hbm, h2a_ref, w3p_ref, obuf, tbuf, sems, tsem):
    j = pl.program_id(0)

    @pl.when(j == 0)
    def _first():
        h1t = jnp.maximum(
            lax.dot_general(w1_ref[...], xe_ref[...],
                            (((0,), (1,)), ((), ())),
                            preferred_element_type=jnp.float32)
            + b1_ref[...], 0.0)
        h2t = jnp.maximum(
            lax.dot_general(w2_ref[...], h1t,
                            (((0,), (0,)), ((), ())),
                            preferred_element_type=jnp.float32)
            + b2_ref[...], 0.0)
        h2a_ref[...] = jnp.zeros((KP, BATCH), jnp.bfloat16)
        h2a_ref[pl.ds(0, H2), :] = h2t.astype(jnp.bfloat16)
        h2a_ref[pl.ds(H2, 1), :] = jnp.ones((1, BATCH), jnp.bfloat16)
        w3p_ref[...] = jnp.zeros((KP, VT), jnp.bfloat16)

    slot = lax.rem(j, NBUF)

    for k in range(NBUF):
        @pl.when(jnp.logical_and(slot == k, j >= NBUF))
        def _reclaim(k=k):
            pltpu.make_async_copy(
                obuf.at[k],
                out_hbm.at[pl.ds((j - NBUF) * VT, VT), :],
                sems.at[k]).wait()

    w3p_ref[pl.ds(0, H2), :] = w3_ref[...].astype(jnp.bfloat16)
    w3p_ref[pl.ds(H2, 1), :] = b3_ref[...].astype(jnp.bfloat16).reshape(1, VT)
    tile = lax.dot_general(w3p_ref[...], h2a_ref[...],
                           (((0,), (0,)), ((), ())),
                           preferred_element_type=jnp.float32)
    obuf[slot] = tile

    for k in range(NBUF):
        @pl.when(jnp.logical_and(slot == k, j < GRID - 1))
        def _issue_full(k=k):
            pltpu.make_async_copy(
                obuf.at[k],
                out_hbm.at[pl.ds(j * VT, VT), :],
                sems.at[k]).start()

    @pl.when(j == GRID - 1)
    def _issue_tail_and_drain():
        tbuf[...] = tile[:TAIL, :]
        pltpu.make_async_copy(
            tbuf,
            out_hbm.at[pl.ds((GRID - 1) * VT, TAIL), :],
            tsem).start()
        for jj in range(GRID - NBUF, GRID - 1):
            s = jj % NBUF
            pltpu.make_async_copy(
                obuf.at[s],
                out_hbm.at[pl.ds(jj * VT, VT), :],
                sems.at[s]).wait()
        pltpu.make_async_copy(
            tbuf,
            out_hbm.at[pl.ds((GRID - 1) * VT, TAIL), :],
            tsem).wait()


_mlp_call = pl.pallas_call(
    _mlp_body,
    grid=(GRID,),
    in_specs=[
        pl.BlockSpec((BATCH, WIN * DIM), lambda j: (0, 0)),
        pl.BlockSpec((WIN * DIM, H1), lambda j: (0, 0)),
        pl.BlockSpec((H1, 1), lambda j: (0, 0)),
        pl.BlockSpec((H1, H2), lambda j: (0, 0)),
        pl.BlockSpec((H2, 1), lambda j: (0, 0)),
        pl.BlockSpec((H2, VT), lambda j: (0, j)),
        pl.BlockSpec((VT,), lambda j: (j,)),
    ],
    out_specs=pl.BlockSpec(memory_space=pl.ANY),
    out_shape=jax.ShapeDtypeStruct((VOCAB, BATCH), jnp.float32),
    scratch_shapes=[
        pltpu.VMEM((KP, BATCH), jnp.bfloat16),
        pltpu.VMEM((KP, VT), jnp.bfloat16),
        pltpu.VMEM((NBUF, VT, BATCH), jnp.float32),
        pltpu.VMEM((TAIL, BATCH), jnp.float32),
        pltpu.SemaphoreType.DMA((NBUF,)),
        pltpu.SemaphoreType.DMA,
    ],
    compiler_params=pltpu.CompilerParams(
        dimension_semantics=("arbitrary",),
    ),
)


def kernel(x, emb, W1, b1, W2, b2, W3, b3):
    idx = x.reshape(_NW * _CH, _CW).astype(jnp.int32)
    rows = _make_sc_gather()(emb, idx)
    xe = rows.reshape(BATCH, WIN * DIM)
    out_t = _mlp_call(xe, W1, b1.reshape(H1, 1), W2, b2.reshape(H2, 1),
                      W3, b3)
    return out_t.T


# speedup vs baseline: 1.0292x; 1.0292x over previous
import functools

import jax
import jax.numpy as jnp
from jax import lax
from jax.experimental import pallas as pl
from jax.experimental.pallas import tpu as pltpu
from jax.experimental.pallas import tpu_sc as plsc

VOCAB = 100000
DIM = 32
WIN = 5
BATCH = 1024
H1 = 80
H2 = 40
NIDX = BATCH * WIN

VT = 2048
GRID = (VOCAB + VT - 1) // VT
NBUF = 4
TAIL = VOCAB - (GRID - 1) * VT
KP = 128

_NC, _NS = 2, 16
_NW = _NC * _NS
_BPW = NIDX // _NW
_CH = 2
_CW = _BPW // _CH


@functools.cache
def _make_sc_gather():
    @functools.partial(
        pl.kernel,
        out_type=jax.ShapeDtypeStruct((NIDX, DIM), jnp.float32),
        mesh=plsc.VectorSubcoreMesh(core_axis_name="c", subcore_axis_name="s"),
        scratch_types=[
            pltpu.VMEM((_CH, _CW), jnp.int32),
            pltpu.VMEM((_CH, _CW, DIM), jnp.float32),
            pltpu.SemaphoreType.DMA,
        ],
        compiler_params=pltpu.CompilerParams(use_tc_tiling_on_sc=False),
    )
    def _sc_gather(table_hbm, idx_hbm, out_hbm, idx_v, rows_v, sem):
        wid = lax.axis_index("s") * _NC + lax.axis_index("c")
        base = wid * _BPW
        copies = []
        for j in range(_CH):
            pltpu.sync_copy(idx_hbm.at[wid * _CH + j], idx_v.at[j])
            copies.append(
                pltpu.async_copy(table_hbm.at[idx_v.at[j]], rows_v.at[j], sem))
        for j in range(_CH):
            copies[j].wait()
            pltpu.sync_copy(rows_v.at[j], out_hbm.at[pl.ds(base + j * _CW, _CW)])

    return _sc_gather


def _mlp_body(xe_ref, w1_ref, b1_ref, w2_ref, b2_ref, w3_ref, b3_ref,
              out_hbm, h2a_ref, w3p_ref, obuf, tbuf, sems, tsem):
    j = pl.program_id(0)

    @pl.when(j == 0)
    def _first():
        h1t = jnp.maximum(
            lax.dot_general(w1_ref[...], xe_ref[...],
                            (((0,), (1,)), ((), ())),
                            preferred_element_type=jnp.float32)
            + b1_ref[...], 0.0)
        h2t = jnp.maximum(
            lax.dot_general(w2_ref[...], h1t,
                            (((0,), (0,)), ((), ())),
                            preferred_element_type=jnp.float32)
            + b2_ref[...], 0.0)
        h2a_ref[...] = jnp.zeros((KP, BATCH), jnp.bfloat16)
        h2a_ref[pl.ds(0, H2), :] = h2t.astype(jnp.bfloat16)
        h2a_ref[pl.ds(H2, 1), :] = jnp.ones((1, BATCH), jnp.bfloat16)
        w3p_ref[...] = jnp.zeros((KP, VT), jnp.bfloat16)

    slot = lax.rem(j, NBUF)

    for k in range(NBUF):
        @pl.when(jnp.logical_and(slot == k, j >= NBUF))
        def _reclaim(k=k):
            pltpu.make_async_copy(
                obuf.at[k],
                out_hbm.at[pl.ds((j - NBUF) * VT, VT), :],
                sems.at[k]).wait()

    w3p_ref[pl.ds(0, H2), :] = w3_ref[...].astype(jnp.bfloat16)
    w3p_ref[pl.ds(H2, 1), :] = b3_ref[...].astype(jnp.bfloat16).reshape(1, VT)
    tile = lax.dot_general(w3p_ref[...], h2a_ref[...],
                           (((0,), (0,)), ((), ())),
                           preferred_element_type=jnp.float32)
    obuf[slot] = tile

    for k in range(NBUF):
        @pl.when(jnp.logical_and(slot == k, j < GRID - 1))
        def _issue_full(k=k):
            pltpu.make_async_copy(
                obuf.at[k],
                out_hbm.at[pl.ds(j * VT, VT), :],
                sems.at[k]).start()

    @pl.when(j == GRID - 1)
    def _issue_tail_and_drain():
        tbuf[...] = tile[:TAIL, :]
        pltpu.make_async_copy(
            tbuf,
            out_hbm.at[pl.ds((GRID - 1) * VT, TAIL), :],
            tsem).start()
        for jj in range(GRID - NBUF, GRID - 1):
            s = jj % NBUF
            pltpu.make_async_copy(
                obuf.at[s],
                out_hbm.at[pl.ds(jj * VT, VT), :],
                sems.at[s]).wait()
        pltpu.make_async_copy(
            tbuf,
            out_hbm.at[pl.ds((GRID - 1) * VT, TAIL), :],
            tsem).wait()


_mlp_call = pl.pallas_call(
    _mlp_body,
    grid=(GRID,),
    in_specs=[
        pl.BlockSpec((BATCH, WIN * DIM), lambda j: (0, 0)),
        pl.BlockSpec((WIN * DIM, H1), lambda j: (0, 0)),
        pl.BlockSpec((H1, 1), lambda j: (0, 0)),
        pl.BlockSpec((H1, H2), lambda j: (0, 0)),
        pl.BlockSpec((H2, 1), lambda j: (0, 0)),
        pl.BlockSpec((H2, VT), lambda j: (0, j)),
        pl.BlockSpec((VT,), lambda j: (j,)),
    ],
    out_specs=pl.BlockSpec(memory_space=pl.ANY),
    out_shape=jax.ShapeDtypeStruct((VOCAB, BATCH), jnp.float32),
    scratch_shapes=[
        pltpu.VMEM((KP, BATCH), jnp.bfloat16),
        pltpu.VMEM((KP, VT), jnp.bfloat16),
        pltpu.VMEM((NBUF, VT, BATCH), jnp.float32),
        pltpu.VMEM((TAIL, BATCH), jnp.float32),
        pltpu.SemaphoreType.DMA((NBUF,)),
        pltpu.SemaphoreType.DMA,
    ],
    compiler_params=pltpu.CompilerParams(
        dimension_semantics=("arbitrary",),
    ),
)


def kernel(x, emb, W1, b1, W2, b2, W3, b3):
    idx = x.reshape(_NW * _CH, _CW).astype(jnp.int32)
    rows = _make_sc_gather()(emb, idx)
    xe = rows.reshape(BATCH, WIN * DIM)
    out_t = _mlp_call(xe, W1, b1.reshape(H1, 1), W2, b2.reshape(H2, 1),
                      W3, b3)
    return out_t.T
